# Initial kernel scaffold; baseline (speedup 1.0000x reference)
import jax
import jax.numpy as jnp
from jax.experimental import pallas as pl


def kernel(x_user, x_item, src_rates, dst_rates, W1_rates, b1_rates, W1_rev, b1_rev, W2_rates, b2_rates, W2_rev, b2_rev):
    raise NotImplementedError("write your pallas kernel here")


# speedup vs baseline: 2.6609x; 2.6609x over previous
import jax
import jax.numpy as jnp
from jax import lax
from jax.experimental import pallas as pl
from jax.experimental.pallas import tpu as pltpu
from jax.experimental.pallas import tpu_sc as plsc

NU = 10000
NE = 160000
FIN = 256
FH = 256
FO = 128
HF = 128

NC = 2
NS = 16
CH = 128
CPT = 79
NEP = NS * CPT * CH
DUMMY = NU
AGR = NU + 16
NPAD = 79 * 128
RB = 80
NRB = NU // RB


def _mesh():
    return plsc.VectorSubcoreMesh(
        core_axis_name="c", subcore_axis_name="s",
        num_cores=NC, num_subcores=NS)


def _deg_body(idx2, deg, idxv, onesv, zb, ob, hist):
    c = lax.axis_index("c")
    s = lax.axis_index("s")
    one = jnp.ones((16,), jnp.float32)
    zero = jnp.zeros((16,), jnp.float32)
    for k in range(CH // 16):
        onesv[pl.ds(16 * k, 16)] = one
    for k in range(RB // 16):
        zb[pl.ds(16 * k, 16)] = zero
    nblk = (NRB - s + NS - 1) // NS

    def zblk(k, carry):
        j = s + k * NS
        pltpu.sync_copy(zb, hist.at[pl.ds(j * RB, RB)])
        return carry

    lax.fori_loop(0, nblk, zblk, None)

    @pl.when(s == 0)
    def _():
        pltpu.sync_copy(zb.at[pl.ds(0, 16)], hist.at[pl.ds(NU, 16)])

    plsc.subcore_barrier()
    pltpu.sync_copy(idx2.at[c, s], idxv)

    def acc(j, carry):
        pltpu.sync_copy(onesv, hist.at[idxv.at[j]], add=True)
        return carry

    lax.fori_loop(0, CPT, acc, None)
    plsc.subcore_barrier()

    def dump(k, carry):
        j = s + k * NS
        pltpu.sync_copy(hist.at[pl.ds(j * RB, RB)], ob)
        pltpu.sync_copy(ob, deg.at[pl.ds(c * NU + j * RB, RB)])
        return carry

    lax.fori_loop(0, nblk, dump, None)


def _degrees(idx2):
    return pl.kernel(
        _deg_body,
        out_type=jax.ShapeDtypeStruct((2 * NU,), jnp.float32),
        mesh=_mesh(),
        scratch_types=[
            pltpu.VMEM((CPT, CH), jnp.int32),
            pltpu.VMEM((CH,), jnp.float32),
            pltpu.VMEM((RB,), jnp.float32),
            pltpu.VMEM((RB,), jnp.float32),
            pltpu.VMEM_SHARED((AGR,), jnp.float32),
        ],
    )(idx2)


def _conv_pass(tab, gidx, sidx, buf, agg, out_slot, s):
    zero = jnp.zeros((16,), jnp.float32)

    def zrow(i, carry):
        for k in range(8):
            buf[i, pl.ds(16 * k, 16)] = zero
        return carry

    lax.fori_loop(0, RB, zrow, None)
    nblk = (NRB - s + NS - 1) // NS

    def zblk(k, carry):
        j = s + k * NS
        pltpu.sync_copy(buf.at[pl.ds(0, RB)], agg.at[pl.ds(j * RB, RB)])
        return carry

    lax.fori_loop(0, nblk, zblk, None)

    @pl.when(s == 0)
    def _():
        pltpu.sync_copy(buf.at[pl.ds(0, 16)], agg.at[pl.ds(NU, 16)])

    plsc.subcore_barrier()

    def conv(j, carry):
        pltpu.sync_copy(tab.at[gidx.at[j]], buf)
        pltpu.sync_copy(buf, agg.at[sidx.at[j]], add=True)
        return carry

    lax.fori_loop(0, CPT, conv, None)
    plsc.subcore_barrier()

    def dump(k, carry):
        j = s + k * NS
        pltpu.sync_copy(agg.at[pl.ds(j * RB, RB)], buf.at[pl.ds(0, RB)])
        pltpu.sync_copy(buf.at[pl.ds(0, RB)], out_slot.at[pl.ds(j * RB, RB)])
        return carry

    lax.fori_loop(0, nblk, dump, None)
    plsc.subcore_barrier()


def _l1_body(hr3, hv3, idx2, agg4, srcv, dstv, buf, agg):
    c = lax.axis_index("c")
    s = lax.axis_index("s")
    pltpu.sync_copy(idx2.at[0, s], srcv)
    pltpu.sync_copy(idx2.at[1, s], dstv)
    for slot, tab3, gidx, sidx in ((0, hr3, srcv, dstv), (1, hv3, dstv, srcv)):
        _conv_pass(tab3.at[c], gidx, sidx, buf, agg, agg4.at[slot, c], s)


def _conv1(hr3, hv3, idx2):
    return pl.kernel(
        _l1_body,
        out_type=jax.ShapeDtypeStruct((2, 2, NU, HF), jnp.float32),
        mesh=_mesh(),
        scratch_types=[
            pltpu.VMEM((CPT, CH), jnp.int32),
            pltpu.VMEM((CPT, CH), jnp.int32),
            pltpu.VMEM((CH, HF), jnp.float32),
            pltpu.VMEM_SHARED((AGR, HF), jnp.float32),
        ],
    )(hr3, hv3, idx2)


def _l2_body(g3, idx2, out2, srcv, dstv, buf, agg):
    c = lax.axis_index("c")
    s = lax.axis_index("s")
    pltpu.sync_copy(idx2.at[c, s], srcv)
    pltpu.sync_copy(idx2.at[1 - c, s], dstv)
    _conv_pass(g3.at[c], srcv, dstv, buf, agg, out2.at[c], s)


def _conv2(g3, idx2):
    return pl.kernel(
        _l2_body,
        out_type=jax.ShapeDtypeStruct((2, NU, FO), jnp.float32),
        mesh=_mesh(),
        scratch_types=[
            pltpu.VMEM((CPT, CH), jnp.int32),
            pltpu.VMEM((CPT, CH), jnp.int32),
            pltpu.VMEM((CH, FO), jnp.float32),
            pltpu.VMEM_SHARED((AGR, FO), jnp.float32),
        ],
    )(g3, idx2)


_RBLK = 128
_NB = NPAD // _RBLK


def _mm1_body(xu, xi, du, di, wr, wv, hr3, hv3):
    nu = lax.rsqrt(jnp.maximum(du[0], 1.0))
    ni = lax.rsqrt(jnp.maximum(di[0], 1.0))
    hr3[...] = jnp.dot(xu[...] * nu, wr[0],
                       preferred_element_type=jnp.float32)[None]
    hv3[...] = jnp.dot(xi[...] * ni, wv[0],
                       preferred_element_type=jnp.float32)[None]


def _mm1(xu, xi, deg3, wr, wv):
    bs_x = pl.BlockSpec((_RBLK, FIN), lambda i, h: (i, 0))
    bs_du = pl.BlockSpec((1, _RBLK, 1), lambda i, h: (0, i, 0))
    bs_di = pl.BlockSpec((1, _RBLK, 1), lambda i, h: (1, i, 0))
    bs_w = pl.BlockSpec((1, FIN, HF), lambda i, h: (h, 0, 0))
    bs_h = pl.BlockSpec((1, _RBLK, HF), lambda i, h: (h, i, 0))
    return pl.pallas_call(
        _mm1_body, grid=(_NB, 2),
        in_specs=[bs_x, bs_x, bs_du, bs_di, bs_w, bs_w],
        out_specs=[bs_h, bs_h],
        out_shape=[jax.ShapeDtypeStruct((2, NPAD, HF), jnp.float32)] * 2,
    )(xu, xi, deg3, deg3, wr, wv)


def _mm2_body(a0, a1, dg, b1, w2, g3):
    n = lax.rsqrt(jnp.maximum(dg[0], 1.0))
    a = jnp.concatenate([a0[0, 0], a1[0, 0]], axis=1)
    t = jnp.maximum(a * n + b1[0], 0.0) * n
    g3[...] = jnp.dot(t, w2[0], preferred_element_type=jnp.float32)[None]


def _mm2(agg4, deg3, b1s, w2s):
    def bs_ak(k):
        return pl.BlockSpec((1, 1, _RBLK, HF),
                            lambda i, r, k=k: (1 - r, k, i, 0))
    bs_d = pl.BlockSpec((1, _RBLK, 1), lambda i, r: (r, i, 0))
    bs_b = pl.BlockSpec((1, 1, FH), lambda i, r: (r, 0, 0))
    bs_w = pl.BlockSpec((1, FH, FO), lambda i, r: (r, 0, 0))
    bs_g = pl.BlockSpec((1, _RBLK, FO), lambda i, r: (r, i, 0))
    return pl.pallas_call(
        _mm2_body, grid=(_NB, 2),
        in_specs=[bs_ak(0), bs_ak(1), bs_d, bs_b, bs_w],
        out_specs=bs_g,
        out_shape=jax.ShapeDtypeStruct((2, NPAD, FO), jnp.float32),
    )(agg4, agg4, deg3, b1s, w2s)


def _fin_body(ag, dg, b2, h2):
    n = lax.rsqrt(jnp.maximum(dg[0], 1.0))
    h2[...] = (ag[0] * n + b2[0])[None]


def _final(out2, deg3, b2s):
    bs_a = pl.BlockSpec((1, _RBLK, FO), lambda i, r: (r, i, 0))
    bs_d = pl.BlockSpec((1, _RBLK, 1), lambda i, r: (1 - r, i, 0))
    bs_b = pl.BlockSpec((1, 1, FO), lambda i, r: (r, 0, 0))
    return pl.pallas_call(
        _fin_body, grid=(pl.cdiv(NU, _RBLK), 2),
        in_specs=[bs_a, bs_d, bs_b],
        out_specs=bs_a,
        out_shape=jax.ShapeDtypeStruct((2, NU, FO), jnp.float32),
    )(out2, deg3, b2s)


def kernel(x_user, x_item, src_rates, dst_rates,
           W1_rates, b1_rates, W1_rev, b1_rev,
           W2_rates, b2_rates, W2_rev, b2_rev):
    pad = jnp.full((NEP - NE,), DUMMY, jnp.int32)
    srcp = jnp.concatenate([src_rates, pad]).reshape(NS, CPT, CH)
    dstp = jnp.concatenate([dst_rates, pad]).reshape(NS, CPT, CH)
    idx2 = jnp.stack([srcp, dstp])
    deg = _degrees(idx2)
    deg3 = deg.reshape(2, NU, 1)
    w1rh = W1_rates.reshape(FIN, 2, HF).transpose(1, 0, 2)
    w1vh = W1_rev.reshape(FIN, 2, HF).transpose(1, 0, 2)
    hr3, hv3 = _mm1(x_user, x_item, deg3, w1rh, w1vh)
    agg4 = _conv1(hr3, hv3, idx2)
    b1s = jnp.stack([b1_rev, b1_rates]).reshape(2, 1, FH)
    w2s = jnp.stack([W2_rates, W2_rev])
    g3 = _mm2(agg4, deg3, b1s, w2s)
    out2 = _conv2(g3, idx2)
    b2s = jnp.stack([b2_rates, b2_rev]).reshape(2, 1, FO)
    h2 = _final(out2, deg3, b2s)
    return (h2[1], h2[0])
